# baseline (speedup 1.0000x reference)
import functools

import jax
import jax.numpy as jnp
from jax import lax
from jax.experimental import pallas as pl
from jax.experimental.pallas import tpu as pltpu
from jax.experimental.pallas import tpu_sc as plsc

N_NODE = 1000000
EMD = 64
B = 16384
LABEL_SMOOTH = 0.1
LAMBDA_GEN = 1e-05

_NC = 2
_NS = 16
_NW = _NC * _NS
_BPW = B // _NW
_CHUNK = 128
_NCH = _BPW // _CHUNK

_BLK = 2048
_NBLK = B // _BLK


def _gather_body(ids_hbm, table_hbm, out_hbm, idx_v, rows_v, sem):
    wid = lax.axis_index("s") * _NC + lax.axis_index("c")
    base = wid * _BPW
    pltpu.sync_copy(ids_hbm.at[wid], idx_v)
    cps = [
        pltpu.async_copy(
            table_hbm.at[idx_v.at[j]],
            rows_v.at[pl.ds(j * _CHUNK, _CHUNK)],
            sem,
        )
        for j in range(_NCH)
    ]
    for cp in cps:
        cp.wait()
    pltpu.sync_copy(rows_v, out_hbm.at[pl.ds(base, _BPW)])


@functools.cache
def _sc_gather_fn():
    return functools.partial(
        pl.kernel,
        out_type=jax.ShapeDtypeStruct((B, 2 * EMD), jnp.float32),
        mesh=plsc.VectorSubcoreMesh(
            core_axis_name="c", subcore_axis_name="s", num_cores=_NC),
        scratch_types=[
            pltpu.VMEM((_NCH, _CHUNK), jnp.int32),
            pltpu.VMEM((_BPW, 2 * EMD), jnp.float32),
            pltpu.SemaphoreType.DMA,
        ],
    )(_gather_body)


def _dense_body(pairs_ref, par_ref, noise_ref, dis_ref, w_ref, b_ref,
                fake0_ref, fake1_ref, sc_ref, acc_ref):
    k = pl.program_id(0)

    @pl.when(k == 0)
    def _init():
        acc_ref[0] = 0.0
        acc_ref[1] = 0.0
        acc_ref[2] = 0.0

    pairs = pairs_ref[...]
    par = par_ref[...]
    emb = jnp.where(par > 0.5, pairs[:, EMD:], pairs[:, :EMD])
    partial_emb = jnp.sum(emb * emb)

    ce = []
    for i in range(2):
        inp = emb + noise_ref[i]
        fake = jnp.dot(inp, w_ref[i], preferred_element_type=jnp.float32)
        fake = fake + b_ref[i]
        fake = jnp.where(fake >= 0, fake, 0.2 * fake)
        if i == 0:
            fake0_ref[...] = fake
        else:
            fake1_ref[...] = fake
        score = jnp.sum(dis_ref[i] * fake, axis=1, keepdims=True)
        ce_el = (jnp.maximum(score, 0.0) - score * (1.0 - LABEL_SMOOTH)
                 + jnp.log(1.0 + jnp.exp(-jnp.abs(score))))
        ce.append(jnp.sum(ce_el))

    acc_ref[0] = acc_ref[0] + ce[0]
    acc_ref[1] = acc_ref[1] + ce[1]
    acc_ref[2] = acc_ref[2] + partial_emb

    @pl.when(k == _NBLK - 1)
    def _fin():
        semb = acc_ref[2]
        w0 = w_ref[0]
        w1 = w_ref[1]
        n0 = (acc_ref[0] / B
              + LAMBDA_GEN * (0.5 * semb + 0.5 * jnp.sum(w0 * w0)))
        n1 = (acc_ref[1] / B
              + LAMBDA_GEN * (0.5 * semb + 0.5 * jnp.sum(w1 * w1)))
        sc_ref[0] = n0 + n1
        sc_ref[1] = n0
        sc_ref[2] = n1


def _dense(pairs, par, noise, dis, w, b3, interpret=False):
    return pl.pallas_call(
        _dense_body,
        grid=(_NBLK,),
        in_specs=[
            pl.BlockSpec((_BLK, 2 * EMD), lambda k: (k, 0)),
            pl.BlockSpec((_BLK, 1), lambda k: (k, 0)),
            pl.BlockSpec((2, _BLK, EMD), lambda k: (0, k, 0)),
            pl.BlockSpec((2, _BLK, EMD), lambda k: (0, k, 0)),
            pl.BlockSpec((2, EMD, EMD), lambda k: (0, 0, 0)),
            pl.BlockSpec((2, 1, EMD), lambda k: (0, 0, 0)),
        ],
        out_specs=[
            pl.BlockSpec((_BLK, EMD), lambda k: (k, 0)),
            pl.BlockSpec((_BLK, EMD), lambda k: (k, 0)),
            pl.BlockSpec(memory_space=pltpu.SMEM),
        ],
        out_shape=[
            jax.ShapeDtypeStruct((B, EMD), jnp.float32),
            jax.ShapeDtypeStruct((B, EMD), jnp.float32),
            jax.ShapeDtypeStruct((3,), jnp.float32),
        ],
        scratch_shapes=[pltpu.SMEM((3,), jnp.float32)],
        compiler_params=pltpu.CompilerParams(
            dimension_semantics=("arbitrary",),
        ),
        interpret=interpret,
    )(pairs, par, noise, dis, w, b3)


def kernel(node_ids, noise_embedding, dis_node_embedding, table, gen_w_1,
           gen_b_1):
    ids = node_ids.astype(jnp.int32)
    pair_ids = (ids >> 1).reshape(_NW, _NCH, _CHUNK)
    par = (ids & 1).astype(jnp.float32).reshape(B, 1)
    table2 = table.reshape(N_NODE // 2, 2 * EMD)
    pairs = _sc_gather_fn()(pair_ids, table2)
    b3 = gen_b_1.reshape(2, 1, EMD)
    fake0, fake1, sc = _dense(pairs, par, noise_embedding,
                              dis_node_embedding, gen_w_1, b3)
    return (sc[0], fake0, fake1, sc[1], sc[2])


# speedup vs baseline: 1.9250x; 1.9250x over previous
import functools

import jax
import jax.numpy as jnp
from jax import lax
from jax.experimental import pallas as pl
from jax.experimental.pallas import tpu as pltpu
from jax.experimental.pallas import tpu_sc as plsc

N_NODE = 1000000
EMD = 64
B = 16384
LABEL_SMOOTH = 0.1
LAMBDA_GEN = 1e-05

NPAIR = 512000

_NC = 2
_NS = 16
_NW = _NC * _NS
_BPW = B // _NW
_CHUNK = 128
_NCH = _BPW // _CHUNK

_BLK = 2048
_NBLK = B // _BLK

_TCOL = 4096
_TGRID = NPAIR // _TCOL
_NHI = N_NODE // _TCOL - 1


def _densify_body(lo_ref, hi_ref, out_ref):
    r = lax.broadcasted_iota(jnp.int32, (EMD, EMD), 0)
    c = lax.broadcasted_iota(jnp.int32, (EMD, EMD), 1)
    ident = jnp.where(r == c, 1.0, 0.0).astype(jnp.float32)
    xt_lo = lax.dot_general(lo_ref[...], ident, (((0,), (0,)), ((), ())),
                            preferred_element_type=jnp.float32)
    xt_hi = lax.dot_general(hi_ref[...], ident, (((0,), (0,)), ((), ())),
                            preferred_element_type=jnp.float32)
    out_ref[...] = jnp.concatenate([xt_lo, xt_hi], axis=1)


def _densify(tableT):
    return pl.pallas_call(
        _densify_body,
        grid=(_TGRID,),
        in_specs=[
            pl.BlockSpec((EMD, _TCOL), lambda k: (0, k)),
            pl.BlockSpec((EMD, _TCOL),
                         lambda k: (0, jnp.minimum(k + _TGRID, _NHI))),
        ],
        out_specs=pl.BlockSpec((_TCOL, 2 * EMD), lambda k: (k, 0)),
        out_shape=jax.ShapeDtypeStruct((NPAIR, 2 * EMD), jnp.float32),
        compiler_params=pltpu.CompilerParams(
            dimension_semantics=("arbitrary",),
        ),
    )(tableT, tableT)


def _gather_body(ids_hbm, table_hbm, out_hbm, idx_v, rows_v, sem):
    wid = lax.axis_index("s") * _NC + lax.axis_index("c")
    base = wid * _BPW
    pltpu.sync_copy(ids_hbm.at[wid], idx_v)
    cps = [
        pltpu.async_copy(
            table_hbm.at[idx_v.at[j]],
            rows_v.at[pl.ds(j * _CHUNK, _CHUNK)],
            sem,
        )
        for j in range(_NCH)
    ]
    for cp in cps:
        cp.wait()
    pltpu.sync_copy(rows_v, out_hbm.at[pl.ds(base, _BPW)])


@functools.cache
def _sc_gather_fn():
    return functools.partial(
        pl.kernel,
        out_type=jax.ShapeDtypeStruct((B, 2 * EMD), jnp.float32),
        mesh=plsc.VectorSubcoreMesh(
            core_axis_name="c", subcore_axis_name="s", num_cores=_NC),
        scratch_types=[
            pltpu.VMEM((_NCH, _CHUNK), jnp.int32),
            pltpu.VMEM((_BPW, 2 * EMD), jnp.float32),
            pltpu.SemaphoreType.DMA,
        ],
    )(_gather_body)


def _dense_body(pairs_ref, par_ref, noise_ref, dis_ref, w_ref, b_ref,
                fake0_ref, fake1_ref, sc_ref, acc_ref):
    k = pl.program_id(0)

    @pl.when(k == 0)
    def _init():
        acc_ref[0] = 0.0
        acc_ref[1] = 0.0
        acc_ref[2] = 0.0

    pairs = pairs_ref[...]
    par = par_ref[...]
    emb = jnp.where(par > 0.5, pairs[:, EMD:], pairs[:, :EMD])
    partial_emb = jnp.sum(emb * emb)

    ce = []
    for i in range(2):
        inp = emb + noise_ref[i]
        fake = jnp.dot(inp, w_ref[i], preferred_element_type=jnp.float32)
        fake = fake + b_ref[i]
        fake = jnp.where(fake >= 0, fake, 0.2 * fake)
        if i == 0:
            fake0_ref[...] = fake
        else:
            fake1_ref[...] = fake
        score = jnp.sum(dis_ref[i] * fake, axis=1, keepdims=True)
        ce_el = (jnp.maximum(score, 0.0) - score * (1.0 - LABEL_SMOOTH)
                 + jnp.log(1.0 + jnp.exp(-jnp.abs(score))))
        ce.append(jnp.sum(ce_el))

    acc_ref[0] = acc_ref[0] + ce[0]
    acc_ref[1] = acc_ref[1] + ce[1]
    acc_ref[2] = acc_ref[2] + partial_emb

    @pl.when(k == _NBLK - 1)
    def _fin():
        semb = acc_ref[2]
        w0 = w_ref[0]
        w1 = w_ref[1]
        n0 = (acc_ref[0] / B
              + LAMBDA_GEN * (0.5 * semb + 0.5 * jnp.sum(w0 * w0)))
        n1 = (acc_ref[1] / B
              + LAMBDA_GEN * (0.5 * semb + 0.5 * jnp.sum(w1 * w1)))
        sc_ref[0] = n0 + n1
        sc_ref[1] = n0
        sc_ref[2] = n1


def _dense(pairs, par, noise, dis, w, b3, interpret=False):
    return pl.pallas_call(
        _dense_body,
        grid=(_NBLK,),
        in_specs=[
            pl.BlockSpec((_BLK, 2 * EMD), lambda k: (k, 0)),
            pl.BlockSpec((_BLK, 1), lambda k: (k, 0)),
            pl.BlockSpec((2, _BLK, EMD), lambda k: (0, k, 0)),
            pl.BlockSpec((2, _BLK, EMD), lambda k: (0, k, 0)),
            pl.BlockSpec((2, EMD, EMD), lambda k: (0, 0, 0)),
            pl.BlockSpec((2, 1, EMD), lambda k: (0, 0, 0)),
        ],
        out_specs=[
            pl.BlockSpec((_BLK, EMD), lambda k: (k, 0)),
            pl.BlockSpec((_BLK, EMD), lambda k: (k, 0)),
            pl.BlockSpec(memory_space=pltpu.SMEM),
        ],
        out_shape=[
            jax.ShapeDtypeStruct((B, EMD), jnp.float32),
            jax.ShapeDtypeStruct((B, EMD), jnp.float32),
            jax.ShapeDtypeStruct((3,), jnp.float32),
        ],
        scratch_shapes=[pltpu.SMEM((3,), jnp.float32)],
        compiler_params=pltpu.CompilerParams(
            dimension_semantics=("arbitrary",),
        ),
        interpret=interpret,
    )(pairs, par, noise, dis, w, b3)


def kernel(node_ids, noise_embedding, dis_node_embedding, table, gen_w_1,
           gen_b_1):
    ids = node_ids.astype(jnp.int32)
    hi = ids >= NPAIR
    pair_ids = jnp.where(hi, ids - NPAIR, ids).reshape(_NW, _NCH, _CHUNK)
    par = hi.astype(jnp.float32).reshape(B, 1)
    pair_table = _densify(table.T)
    pairs = _sc_gather_fn()(pair_ids, pair_table)
    b3 = gen_b_1.reshape(2, 1, EMD)
    fake0, fake1, sc = _dense(pairs, par, noise_embedding,
                              dis_node_embedding, gen_w_1, b3)
    return (sc[0], fake0, fake1, sc[1], sc[2])


# speedup vs baseline: 2.0558x; 1.0679x over previous
import functools

import jax
import jax.numpy as jnp
from jax import lax
from jax.experimental import pallas as pl
from jax.experimental.pallas import tpu as pltpu
from jax.experimental.pallas import tpu_sc as plsc

N_NODE = 1000000
EMD = 64
B = 16384
LABEL_SMOOTH = 0.1
LAMBDA_GEN = 1e-05

NPAIR = 512000

_NC = 2
_NS = 16
_NW = _NC * _NS
_BPW = B // _NW
_CHUNK = 128
_NCH = _BPW // _CHUNK

_BLK = 2048
_NBLK = B // _BLK

_TCOL = 4096
_NLO = NPAIR // _TCOL
_TGRID = _NLO + 1
_TAILSTART = (N_NODE // _TCOL) * _TCOL
_TAILN = N_NODE - _TAILSTART
_NHIB = _TAILSTART // _TCOL - 1
PTAB = _TGRID * _TCOL


def _densify_body(lo_ref, hi_ref, tail_ref, out_ref):
    k = pl.program_id(0)

    @pl.when(k < _NLO)
    def _main():
        out_ref[...] = jnp.concatenate([lo_ref[...].T, hi_ref[...].T],
                                       axis=1)

    @pl.when(k == _NLO)
    def _tail():
        out_ref[pl.ds(0, _TAILN), pl.ds(0, EMD)] = tail_ref[...]


def _densify(tableT, tail):
    return pl.pallas_call(
        _densify_body,
        grid=(_TGRID,),
        in_specs=[
            pl.BlockSpec((EMD, _TCOL),
                         lambda k: (0, jnp.minimum(k, _NLO - 1))),
            pl.BlockSpec((EMD, _TCOL),
                         lambda k: (0, jnp.minimum(k + _NLO, _NHIB))),
            pl.BlockSpec((_TAILN, EMD), lambda k: (0, 0)),
        ],
        out_specs=pl.BlockSpec((_TCOL, 2 * EMD), lambda k: (k, 0)),
        out_shape=jax.ShapeDtypeStruct((PTAB, 2 * EMD), jnp.float32),
        compiler_params=pltpu.CompilerParams(
            dimension_semantics=("arbitrary",),
        ),
    )(tableT, tableT, tail)


def _gather_body(ids_hbm, table_hbm, out_hbm, idx_v, rows_v, sem):
    wid = lax.axis_index("s") * _NC + lax.axis_index("c")
    base = wid * _BPW
    pltpu.sync_copy(ids_hbm.at[wid], idx_v)
    cps = [
        pltpu.async_copy(
            table_hbm.at[idx_v.at[j]],
            rows_v.at[pl.ds(j * _CHUNK, _CHUNK)],
            sem,
        )
        for j in range(_NCH)
    ]
    for cp in cps:
        cp.wait()
    pltpu.sync_copy(rows_v, out_hbm.at[pl.ds(base, _BPW)])


@functools.cache
def _sc_gather_fn():
    return functools.partial(
        pl.kernel,
        out_type=jax.ShapeDtypeStruct((B, 2 * EMD), jnp.float32),
        mesh=plsc.VectorSubcoreMesh(
            core_axis_name="c", subcore_axis_name="s", num_cores=_NC),
        scratch_types=[
            pltpu.VMEM((_NCH, _CHUNK), jnp.int32),
            pltpu.VMEM((_BPW, 2 * EMD), jnp.float32),
            pltpu.SemaphoreType.DMA,
        ],
    )(_gather_body)


def _dense_body(pairs_ref, par_ref, noise_ref, dis_ref, w_ref, b_ref,
                fake0_ref, fake1_ref, sc_ref, acc_ref):
    k = pl.program_id(0)

    @pl.when(k == 0)
    def _init():
        acc_ref[0] = 0.0
        acc_ref[1] = 0.0
        acc_ref[2] = 0.0

    pairs = pairs_ref[...]
    par = par_ref[...]
    lane = lax.broadcasted_iota(jnp.int32, (_BLK, 2 * EMD), 1)
    keep = (lane < EMD) == (par < 0.5)
    m = jnp.where(keep, pairs, 0.0)
    partial_emb = jnp.sum(m * m)

    ce = []
    for i in range(2):
        w2 = jnp.concatenate([w_ref[i], w_ref[i]], axis=0)
        embw = jnp.dot(m, w2, preferred_element_type=jnp.float32)
        noisew = jnp.dot(noise_ref[i], w_ref[i],
                         preferred_element_type=jnp.float32)
        fake = embw + noisew + b_ref[i]
        fake = jnp.where(fake >= 0, fake, 0.2 * fake)
        if i == 0:
            fake0_ref[...] = fake.T
        else:
            fake1_ref[...] = fake.T
        score = jnp.sum(dis_ref[i] * fake, axis=1, keepdims=True)
        ce_el = (jnp.maximum(score, 0.0) - score * (1.0 - LABEL_SMOOTH)
                 + jnp.log(1.0 + jnp.exp(-jnp.abs(score))))
        ce.append(jnp.sum(ce_el))

    acc_ref[0] = acc_ref[0] + ce[0]
    acc_ref[1] = acc_ref[1] + ce[1]
    acc_ref[2] = acc_ref[2] + partial_emb

    @pl.when(k == _NBLK - 1)
    def _fin():
        semb = acc_ref[2]
        w0 = w_ref[0]
        w1 = w_ref[1]
        n0 = (acc_ref[0] / B
              + LAMBDA_GEN * (0.5 * semb + 0.5 * jnp.sum(w0 * w0)))
        n1 = (acc_ref[1] / B
              + LAMBDA_GEN * (0.5 * semb + 0.5 * jnp.sum(w1 * w1)))
        sc_ref[0] = n0 + n1
        sc_ref[1] = n0
        sc_ref[2] = n1


def _dense(pairs, par, noise, dis, w, b3, interpret=False):
    return pl.pallas_call(
        _dense_body,
        grid=(_NBLK,),
        in_specs=[
            pl.BlockSpec((_BLK, 2 * EMD), lambda k: (k, 0)),
            pl.BlockSpec((_BLK, 1), lambda k: (k, 0)),
            pl.BlockSpec((2, _BLK, EMD), lambda k: (0, k, 0)),
            pl.BlockSpec((2, _BLK, EMD), lambda k: (0, k, 0)),
            pl.BlockSpec((2, EMD, EMD), lambda k: (0, 0, 0)),
            pl.BlockSpec((2, 1, EMD), lambda k: (0, 0, 0)),
        ],
        out_specs=[
            pl.BlockSpec((EMD, _BLK), lambda k: (0, k)),
            pl.BlockSpec((EMD, _BLK), lambda k: (0, k)),
            pl.BlockSpec(memory_space=pltpu.SMEM),
        ],
        out_shape=[
            jax.ShapeDtypeStruct((EMD, B), jnp.float32),
            jax.ShapeDtypeStruct((EMD, B), jnp.float32),
            jax.ShapeDtypeStruct((3,), jnp.float32),
        ],
        scratch_shapes=[pltpu.SMEM((3,), jnp.float32)],
        compiler_params=pltpu.CompilerParams(
            dimension_semantics=("arbitrary",),
        ),
        interpret=interpret,
    )(pairs, par, noise, dis, w, b3)


def kernel(node_ids, noise_embedding, dis_node_embedding, table, gen_w_1,
           gen_b_1):
    ids = node_ids.astype(jnp.int32)
    tail = ids >= _TAILSTART
    hi = (ids >= NPAIR) & ~tail
    pid = jnp.where(tail, ids - _TAILSTART + NPAIR,
                    jnp.where(hi, ids - NPAIR, ids))
    pair_ids = pid.reshape(_NW, _NCH, _CHUNK)
    par = hi.astype(jnp.float32).reshape(B, 1)
    pair_table = _densify(
        table.T, lax.slice(table, (_TAILSTART, 0), (N_NODE, EMD)))
    pairs = _sc_gather_fn()(pair_ids, pair_table)
    b3 = gen_b_1.reshape(2, 1, EMD)
    fake0t, fake1t, sc = _dense(pairs, par, noise_embedding,
                                dis_node_embedding, gen_w_1, b3)
    return (sc[0], fake0t.T, fake1t.T, sc[1], sc[2])


# speedup vs baseline: 2.2429x; 1.0910x over previous
import functools

import jax
import jax.numpy as jnp
from jax import lax
from jax.experimental import pallas as pl
from jax.experimental.pallas import tpu as pltpu
from jax.experimental.pallas import tpu_sc as plsc

N_NODE = 1000000
EMD = 64
B = 16384
LABEL_SMOOTH = 0.1
LAMBDA_GEN = 1e-05

NPAIR = 512000

_NC = 2
_NS = 16
_NW = _NC * _NS
_BPW = B // _NW
_CHUNK = 128
_NCH = _BPW // _CHUNK

_BLK = 2048
_NBLK = B // _BLK

_TCOL = 4096
_NLO = NPAIR // _TCOL
_TGRID = _NLO + 1
_TAILSTART = (N_NODE // _TCOL) * _TCOL
_TAILN = N_NODE - _TAILSTART
_NHIB = _TAILSTART // _TCOL - 1
PTAB = _TGRID * _TCOL


def _densify_body(lo_ref, hi_ref, tail_ref, out_ref):
    k = pl.program_id(0)

    @pl.when(k < _NLO)
    def _main():
        out_ref[...] = jnp.concatenate([lo_ref[...].T, hi_ref[...].T],
                                       axis=1)

    @pl.when(k == _NLO)
    def _tail():
        out_ref[pl.ds(0, _TAILN), pl.ds(0, EMD)] = tail_ref[...]


def _densify(tableT, tail):
    return pl.pallas_call(
        _densify_body,
        grid=(_TGRID,),
        in_specs=[
            pl.BlockSpec((EMD, _TCOL),
                         lambda k: (0, jnp.minimum(k, _NLO - 1))),
            pl.BlockSpec((EMD, _TCOL),
                         lambda k: (0, jnp.minimum(k + _NLO, _NHIB))),
            pl.BlockSpec((_TAILN, EMD), lambda k: (0, 0)),
        ],
        out_specs=pl.BlockSpec((_TCOL, 2 * EMD), lambda k: (k, 0)),
        out_shape=jax.ShapeDtypeStruct((PTAB, 2 * EMD), jnp.float32),
        compiler_params=pltpu.CompilerParams(
            dimension_semantics=("arbitrary",),
        ),
    )(tableT, tableT, tail)


def _gather_body(ids_hbm, table_hbm, out_hbm, idx_v, rows_v, sem):
    wid = lax.axis_index("s") * _NC + lax.axis_index("c")
    base = wid * _BPW
    pltpu.sync_copy(ids_hbm.at[wid], idx_v)
    cps = [
        pltpu.async_copy(
            table_hbm.at[idx_v.at[j]],
            rows_v.at[pl.ds(j * _CHUNK, _CHUNK)],
            sem,
        )
        for j in range(_NCH)
    ]
    for cp in cps:
        cp.wait()
    pltpu.sync_copy(rows_v, out_hbm.at[pl.ds(base, _BPW)])


@functools.cache
def _sc_gather_fn():
    return functools.partial(
        pl.kernel,
        out_type=jax.ShapeDtypeStruct((B, 2 * EMD), jnp.float32),
        mesh=plsc.VectorSubcoreMesh(
            core_axis_name="c", subcore_axis_name="s", num_cores=_NC),
        scratch_types=[
            pltpu.VMEM((_NCH, _CHUNK), jnp.int32),
            pltpu.VMEM((_BPW, 2 * EMD), jnp.float32),
            pltpu.SemaphoreType.DMA,
        ],
    )(_gather_body)


def _dense_body(pairs_ref, par_ref, noise_ref, dis_ref, w_ref, b_ref,
                fake0_ref, fake1_ref, sc_ref, acc_ref):
    k = pl.program_id(0)

    @pl.when(k == 0)
    def _init():
        acc_ref[0] = 0.0
        acc_ref[1] = 0.0
        acc_ref[2] = 0.0

    pairs = pairs_ref[...]
    par = par_ref[...]
    lane = lax.broadcasted_iota(jnp.int32, (_BLK, 2 * EMD), 1)
    keep = (lane < EMD) == (par < 0.5)
    m = jnp.where(keep, pairs, 0.0)
    partial_emb = jnp.sum(m * m)

    ce = []
    for i in range(2):
        w2 = jnp.concatenate([w_ref[i], w_ref[i]], axis=0)
        embw = jnp.dot(m, w2, preferred_element_type=jnp.float32)
        noisew = lax.dot_general(noise_ref[i], w_ref[i],
                                 (((0,), (0,)), ((), ())),
                                 preferred_element_type=jnp.float32)
        fake = embw + noisew + b_ref[i]
        fake = jnp.where(fake >= 0, fake, 0.2 * fake)
        faket = fake.T
        if i == 0:
            fake0_ref[...] = faket
        else:
            fake1_ref[...] = faket
        score = jnp.sum(dis_ref[i] * faket, axis=0, keepdims=True)
        ce_el = (jnp.maximum(score, 0.0) - score * (1.0 - LABEL_SMOOTH)
                 + jnp.log(1.0 + jnp.exp(-jnp.abs(score))))
        ce.append(jnp.sum(ce_el))

    acc_ref[0] = acc_ref[0] + ce[0]
    acc_ref[1] = acc_ref[1] + ce[1]
    acc_ref[2] = acc_ref[2] + partial_emb

    @pl.when(k == _NBLK - 1)
    def _fin():
        semb = acc_ref[2]
        w0 = w_ref[0]
        w1 = w_ref[1]
        n0 = (acc_ref[0] / B
              + LAMBDA_GEN * (0.5 * semb + 0.5 * jnp.sum(w0 * w0)))
        n1 = (acc_ref[1] / B
              + LAMBDA_GEN * (0.5 * semb + 0.5 * jnp.sum(w1 * w1)))
        sc_ref[0] = n0 + n1
        sc_ref[1] = n0
        sc_ref[2] = n1


def _dense(pairs, par, noise, dis, w, b3, interpret=False):
    return pl.pallas_call(
        _dense_body,
        grid=(_NBLK,),
        in_specs=[
            pl.BlockSpec((_BLK, 2 * EMD), lambda k: (k, 0)),
            pl.BlockSpec((_BLK, 1), lambda k: (k, 0)),
            pl.BlockSpec((2, EMD, _BLK), lambda k: (0, 0, k)),
            pl.BlockSpec((2, EMD, _BLK), lambda k: (0, 0, k)),
            pl.BlockSpec((2, EMD, EMD), lambda k: (0, 0, 0)),
            pl.BlockSpec((2, 1, EMD), lambda k: (0, 0, 0)),
        ],
        out_specs=[
            pl.BlockSpec((EMD, _BLK), lambda k: (0, k)),
            pl.BlockSpec((EMD, _BLK), lambda k: (0, k)),
            pl.BlockSpec(memory_space=pltpu.SMEM),
        ],
        out_shape=[
            jax.ShapeDtypeStruct((EMD, B), jnp.float32),
            jax.ShapeDtypeStruct((EMD, B), jnp.float32),
            jax.ShapeDtypeStruct((3,), jnp.float32),
        ],
        scratch_shapes=[pltpu.SMEM((3,), jnp.float32)],
        compiler_params=pltpu.CompilerParams(
            dimension_semantics=("arbitrary",),
        ),
        interpret=interpret,
    )(pairs, par, noise, dis, w, b3)


def kernel(node_ids, noise_embedding, dis_node_embedding, table, gen_w_1,
           gen_b_1):
    ids = node_ids.astype(jnp.int32)
    tail = ids >= _TAILSTART
    hi = (ids >= NPAIR) & ~tail
    pid = jnp.where(tail, ids - _TAILSTART + NPAIR,
                    jnp.where(hi, ids - NPAIR, ids))
    pair_ids = pid.reshape(_NW, _NCH, _CHUNK)
    par = hi.astype(jnp.float32).reshape(B, 1)
    pair_table = _densify(
        table.T, lax.slice(table, (_TAILSTART, 0), (N_NODE, EMD)))
    pairs = _sc_gather_fn()(pair_ids, pair_table)
    b3 = gen_b_1.reshape(2, 1, EMD)
    fake0t, fake1t, sc = _dense(pairs, par,
                                jnp.swapaxes(noise_embedding, 1, 2),
                                jnp.swapaxes(dis_node_embedding, 1, 2),
                                gen_w_1, b3)
    return (sc[0], fake0t.T, fake1t.T, sc[1], sc[2])


# speedup vs baseline: 2.4733x; 1.1028x over previous
import functools

import jax
import jax.numpy as jnp
from jax import lax
from jax.experimental import pallas as pl
from jax.experimental.pallas import tpu as pltpu
from jax.experimental.pallas import tpu_sc as plsc

N_NODE = 1000000
EMD = 64
B = 16384
LABEL_SMOOTH = 0.1
LAMBDA_GEN = 1e-05

NPAIR = 524288

_NC = 2
_NS = 16
_NW = _NC * _NS
_BPW = B // _NW
_CHUNK = 128
_NCH = _BPW // _CHUNK

_BLK = 2048
_NBLK = B // _BLK

_TCOL = 8192
_NLO = NPAIR // _TCOL
_TGRID = _NLO + 1
_TAILSTART = (N_NODE // _TCOL) * _TCOL
_TAILN = N_NODE - _TAILSTART
_NHIB = _TAILSTART // _TCOL - 1
PTAB = _TGRID * _TCOL


def _densify_body(lo_ref, hi_ref, tail_ref, out_ref):
    k = pl.program_id(0)

    @pl.when(k < _NLO)
    def _main():
        out_ref[...] = jnp.concatenate([lo_ref[...].T, hi_ref[...].T],
                                       axis=1)

    @pl.when(k == _NLO)
    def _tail():
        out_ref[pl.ds(0, _TAILN), pl.ds(0, EMD)] = tail_ref[...]


def _densify(tableT, tail):
    return pl.pallas_call(
        _densify_body,
        grid=(_TGRID,),
        in_specs=[
            pl.BlockSpec((EMD, _TCOL),
                         lambda k: (0, jnp.minimum(k, _NLO - 1))),
            pl.BlockSpec((EMD, _TCOL),
                         lambda k: (0, jnp.minimum(k + _NLO, _NHIB))),
            pl.BlockSpec((_TAILN, EMD), lambda k: (0, 0)),
        ],
        out_specs=pl.BlockSpec((_TCOL, 2 * EMD), lambda k: (k, 0)),
        out_shape=jax.ShapeDtypeStruct((PTAB, 2 * EMD), jnp.float32),
        compiler_params=pltpu.CompilerParams(
            dimension_semantics=("arbitrary",),
        ),
    )(tableT, tableT, tail)


def _gather_body(ids_hbm, table_hbm, out_hbm, idx_v, rows_v, sem):
    wid = lax.axis_index("s") * _NC + lax.axis_index("c")
    base = wid * _BPW
    pltpu.sync_copy(ids_hbm.at[wid], idx_v)
    cps = [
        pltpu.async_copy(
            table_hbm.at[idx_v.at[j]],
            rows_v.at[pl.ds(j * _CHUNK, _CHUNK)],
            sem,
        )
        for j in range(_NCH)
    ]
    for cp in cps:
        cp.wait()
    pltpu.sync_copy(rows_v, out_hbm.at[pl.ds(base, _BPW)])


@functools.cache
def _sc_gather_fn():
    return functools.partial(
        pl.kernel,
        out_type=jax.ShapeDtypeStruct((B, 2 * EMD), jnp.float32),
        mesh=plsc.VectorSubcoreMesh(
            core_axis_name="c", subcore_axis_name="s", num_cores=_NC),
        scratch_types=[
            pltpu.VMEM((_NCH, _CHUNK), jnp.int32),
            pltpu.VMEM((_BPW, 2 * EMD), jnp.float32),
            pltpu.SemaphoreType.DMA,
        ],
    )(_gather_body)


def _dense_body(pairs_ref, par_ref, noise_ref, dis_ref, w_ref, b_ref,
                fake0_ref, fake1_ref, sc_ref, acc_ref):
    k = pl.program_id(0)

    @pl.when(k == 0)
    def _init():
        acc_ref[0] = 0.0
        acc_ref[1] = 0.0
        acc_ref[2] = 0.0

    pairs = pairs_ref[...]
    par = par_ref[...]
    lane = lax.broadcasted_iota(jnp.int32, (_BLK, 2 * EMD), 1)
    keep = (lane < EMD) == (par < 0.5)
    m = jnp.where(keep, pairs, 0.0)
    partial_emb = jnp.sum(m * m)

    ce = []
    for i in range(2):
        w2 = jnp.concatenate([w_ref[i], w_ref[i]], axis=0)
        embw = jnp.dot(m, w2, preferred_element_type=jnp.float32)
        noisew = lax.dot_general(noise_ref[i], w_ref[i],
                                 (((0,), (0,)), ((), ())),
                                 preferred_element_type=jnp.float32)
        fake = embw + noisew + b_ref[i]
        fake = jnp.where(fake >= 0, fake, 0.2 * fake)
        faket = fake.T
        if i == 0:
            fake0_ref[...] = faket
        else:
            fake1_ref[...] = faket
        score = jnp.sum(dis_ref[i] * faket, axis=0, keepdims=True)
        ce_el = (jnp.maximum(score, 0.0) - score * (1.0 - LABEL_SMOOTH)
                 + jnp.log(1.0 + jnp.exp(-jnp.abs(score))))
        ce.append(jnp.sum(ce_el))

    acc_ref[0] = acc_ref[0] + ce[0]
    acc_ref[1] = acc_ref[1] + ce[1]
    acc_ref[2] = acc_ref[2] + partial_emb

    @pl.when(k == _NBLK - 1)
    def _fin():
        semb = acc_ref[2]
        w0 = w_ref[0]
        w1 = w_ref[1]
        n0 = (acc_ref[0] / B
              + LAMBDA_GEN * (0.5 * semb + 0.5 * jnp.sum(w0 * w0)))
        n1 = (acc_ref[1] / B
              + LAMBDA_GEN * (0.5 * semb + 0.5 * jnp.sum(w1 * w1)))
        sc_ref[0] = n0 + n1
        sc_ref[1] = n0
        sc_ref[2] = n1


def _dense(pairs, par, noise, dis, w, b3, interpret=False):
    return pl.pallas_call(
        _dense_body,
        grid=(_NBLK,),
        in_specs=[
            pl.BlockSpec((_BLK, 2 * EMD), lambda k: (k, 0)),
            pl.BlockSpec((_BLK, 1), lambda k: (k, 0)),
            pl.BlockSpec((2, EMD, _BLK), lambda k: (0, 0, k)),
            pl.BlockSpec((2, EMD, _BLK), lambda k: (0, 0, k)),
            pl.BlockSpec((2, EMD, EMD), lambda k: (0, 0, 0)),
            pl.BlockSpec((2, 1, EMD), lambda k: (0, 0, 0)),
        ],
        out_specs=[
            pl.BlockSpec((EMD, _BLK), lambda k: (0, k)),
            pl.BlockSpec((EMD, _BLK), lambda k: (0, k)),
            pl.BlockSpec(memory_space=pltpu.SMEM),
        ],
        out_shape=[
            jax.ShapeDtypeStruct((EMD, B), jnp.float32),
            jax.ShapeDtypeStruct((EMD, B), jnp.float32),
            jax.ShapeDtypeStruct((3,), jnp.float32),
        ],
        scratch_shapes=[pltpu.SMEM((3,), jnp.float32)],
        compiler_params=pltpu.CompilerParams(
            dimension_semantics=("arbitrary",),
        ),
        interpret=interpret,
    )(pairs, par, noise, dis, w, b3)


def kernel(node_ids, noise_embedding, dis_node_embedding, table, gen_w_1,
           gen_b_1):
    ids = node_ids.astype(jnp.int32)
    tail = ids >= _TAILSTART
    hi = (ids >= NPAIR) & ~tail
    pid = jnp.where(tail, ids - _TAILSTART + NPAIR,
                    jnp.where(hi, ids - NPAIR, ids))
    pair_ids = pid.reshape(_NW, _NCH, _CHUNK)
    par = hi.astype(jnp.float32).reshape(B, 1)
    pair_table = _densify(
        table.T, lax.slice(table, (_TAILSTART, 0), (N_NODE, EMD)))
    pairs = _sc_gather_fn()(pair_ids, pair_table)
    b3 = gen_b_1.reshape(2, 1, EMD)
    fake0t, fake1t, sc = _dense(pairs, par,
                                jnp.swapaxes(noise_embedding, 1, 2),
                                jnp.swapaxes(dis_node_embedding, 1, 2),
                                gen_w_1, b3)
    return (sc[0], fake0t.T, fake1t.T, sc[1], sc[2])


# speedup vs baseline: 2.5807x; 1.0434x over previous
import functools

import jax
import jax.numpy as jnp
from jax import lax
from jax.experimental import pallas as pl
from jax.experimental.pallas import tpu as pltpu
from jax.experimental.pallas import tpu_sc as plsc

N_NODE = 1000000
EMD = 64
B = 16384
LABEL_SMOOTH = 0.1
LAMBDA_GEN = 1e-05

NPAIR = 524288

_NC = 2
_NS = 16
_NW = _NC * _NS
_BPW = B // _NW
_CHUNK = 128
_NCH = _BPW // _CHUNK

_BLK = 2048
_NBLK = B // _BLK

_TCOL = 16384
_NLO = NPAIR // _TCOL
_TGRID = _NLO + 1
_TAILSTART = (N_NODE // _TCOL) * _TCOL
_TAILN = N_NODE - _TAILSTART
_NHIB = _TAILSTART // _TCOL - 1
PTAB = _TGRID * _TCOL


def _densify_body(lo_ref, hi_ref, tail_ref, out_ref):
    k = pl.program_id(0)

    @pl.when(k < _NLO)
    def _main():
        out_ref[...] = jnp.concatenate([lo_ref[...].T, hi_ref[...].T],
                                       axis=1)

    @pl.when(k == _NLO)
    def _tail():
        out_ref[pl.ds(0, _TAILN), pl.ds(0, EMD)] = tail_ref[...]


def _densify(tableT, tail):
    return pl.pallas_call(
        _densify_body,
        grid=(_TGRID,),
        in_specs=[
            pl.BlockSpec((EMD, _TCOL),
                         lambda k: (0, jnp.minimum(k, _NLO - 1))),
            pl.BlockSpec((EMD, _TCOL),
                         lambda k: (0, jnp.minimum(k + _NLO, _NHIB))),
            pl.BlockSpec((_TAILN, EMD), lambda k: (0, 0)),
        ],
        out_specs=pl.BlockSpec((_TCOL, 2 * EMD), lambda k: (k, 0)),
        out_shape=jax.ShapeDtypeStruct((PTAB, 2 * EMD), jnp.float32),
        compiler_params=pltpu.CompilerParams(
            dimension_semantics=("arbitrary",),
        ),
    )(tableT, tableT, tail)


def _gather_body(ids_hbm, table_hbm, out_hbm, idx_v, rows_v, sem):
    wid = lax.axis_index("s") * _NC + lax.axis_index("c")
    base = wid * _BPW
    pltpu.sync_copy(ids_hbm.at[wid], idx_v)
    cps = [
        pltpu.async_copy(
            table_hbm.at[idx_v.at[j]],
            rows_v.at[pl.ds(j * _CHUNK, _CHUNK)],
            sem,
        )
        for j in range(_NCH)
    ]
    for cp in cps:
        cp.wait()
    pltpu.sync_copy(rows_v, out_hbm.at[pl.ds(base, _BPW)])


@functools.cache
def _sc_gather_fn():
    return functools.partial(
        pl.kernel,
        out_type=jax.ShapeDtypeStruct((B, 2 * EMD), jnp.float32),
        mesh=plsc.VectorSubcoreMesh(
            core_axis_name="c", subcore_axis_name="s", num_cores=_NC),
        scratch_types=[
            pltpu.VMEM((_NCH, _CHUNK), jnp.int32),
            pltpu.VMEM((_BPW, 2 * EMD), jnp.float32),
            pltpu.SemaphoreType.DMA,
        ],
    )(_gather_body)


def _dense_body(pairs_ref, par_ref, noise_ref, dis_ref, w_ref, b_ref,
                fake0_ref, fake1_ref, sc_ref, acc_ref):
    k = pl.program_id(0)

    @pl.when(k == 0)
    def _init():
        acc_ref[0] = 0.0
        acc_ref[1] = 0.0
        acc_ref[2] = 0.0

    pairs = pairs_ref[...]
    par = par_ref[...]
    lane = lax.broadcasted_iota(jnp.int32, (_BLK, 2 * EMD), 1)
    keep = (lane < EMD) == (par < 0.5)
    m = jnp.where(keep, pairs, 0.0)
    partial_emb = jnp.sum(m * m)

    ce = []
    for i in range(2):
        w2 = jnp.concatenate([w_ref[i], w_ref[i]], axis=0)
        embw = jnp.dot(m, w2, preferred_element_type=jnp.float32)
        noisew = lax.dot_general(noise_ref[i], w_ref[i],
                                 (((0,), (0,)), ((), ())),
                                 preferred_element_type=jnp.float32)
        fake = embw + noisew + b_ref[i]
        fake = jnp.where(fake >= 0, fake, 0.2 * fake)
        faket = fake.T
        if i == 0:
            fake0_ref[...] = faket
        else:
            fake1_ref[...] = faket
        score = jnp.sum(dis_ref[i] * faket, axis=0, keepdims=True)
        ce_el = (jnp.maximum(score, 0.0) - score * (1.0 - LABEL_SMOOTH)
                 + jnp.log(1.0 + jnp.exp(-jnp.abs(score))))
        ce.append(jnp.sum(ce_el))

    acc_ref[0] = acc_ref[0] + ce[0]
    acc_ref[1] = acc_ref[1] + ce[1]
    acc_ref[2] = acc_ref[2] + partial_emb

    @pl.when(k == _NBLK - 1)
    def _fin():
        semb = acc_ref[2]
        w0 = w_ref[0]
        w1 = w_ref[1]
        n0 = (acc_ref[0] / B
              + LAMBDA_GEN * (0.5 * semb + 0.5 * jnp.sum(w0 * w0)))
        n1 = (acc_ref[1] / B
              + LAMBDA_GEN * (0.5 * semb + 0.5 * jnp.sum(w1 * w1)))
        sc_ref[0] = n0 + n1
        sc_ref[1] = n0
        sc_ref[2] = n1


def _dense(pairs, par, noise, dis, w, b3, interpret=False):
    return pl.pallas_call(
        _dense_body,
        grid=(_NBLK,),
        in_specs=[
            pl.BlockSpec((_BLK, 2 * EMD), lambda k: (k, 0)),
            pl.BlockSpec((_BLK, 1), lambda k: (k, 0)),
            pl.BlockSpec((2, EMD, _BLK), lambda k: (0, 0, k)),
            pl.BlockSpec((2, EMD, _BLK), lambda k: (0, 0, k)),
            pl.BlockSpec((2, EMD, EMD), lambda k: (0, 0, 0)),
            pl.BlockSpec((2, 1, EMD), lambda k: (0, 0, 0)),
        ],
        out_specs=[
            pl.BlockSpec((EMD, _BLK), lambda k: (0, k)),
            pl.BlockSpec((EMD, _BLK), lambda k: (0, k)),
            pl.BlockSpec(memory_space=pltpu.SMEM),
        ],
        out_shape=[
            jax.ShapeDtypeStruct((EMD, B), jnp.float32),
            jax.ShapeDtypeStruct((EMD, B), jnp.float32),
            jax.ShapeDtypeStruct((3,), jnp.float32),
        ],
        scratch_shapes=[pltpu.SMEM((3,), jnp.float32)],
        compiler_params=pltpu.CompilerParams(
            dimension_semantics=("arbitrary",),
        ),
        interpret=interpret,
    )(pairs, par, noise, dis, w, b3)


def kernel(node_ids, noise_embedding, dis_node_embedding, table, gen_w_1,
           gen_b_1):
    ids = node_ids.astype(jnp.int32)
    tail = ids >= _TAILSTART
    hi = (ids >= NPAIR) & ~tail
    pid = jnp.where(tail, ids - _TAILSTART + NPAIR,
                    jnp.where(hi, ids - NPAIR, ids))
    pair_ids = pid.reshape(_NW, _NCH, _CHUNK)
    par = hi.astype(jnp.float32).reshape(B, 1)
    pair_table = _densify(
        table.T, lax.slice(table, (_TAILSTART, 0), (N_NODE, EMD)))
    pairs = _sc_gather_fn()(pair_ids, pair_table)
    b3 = gen_b_1.reshape(2, 1, EMD)
    fake0t, fake1t, sc = _dense(pairs, par,
                                jnp.swapaxes(noise_embedding, 1, 2),
                                jnp.swapaxes(dis_node_embedding, 1, 2),
                                gen_w_1, b3)
    return (sc[0], fake0t.T, fake1t.T, sc[1], sc[2])


# speedup vs baseline: 2.5983x; 1.0068x over previous
import functools

import jax
import jax.numpy as jnp
from jax import lax
from jax.experimental import pallas as pl
from jax.experimental.pallas import tpu as pltpu
from jax.experimental.pallas import tpu_sc as plsc

N_NODE = 1000000
EMD = 64
B = 16384
LABEL_SMOOTH = 0.1
LAMBDA_GEN = 1e-05

NPAIR = 524288

_NC = 2
_NS = 16
_NW = _NC * _NS
_BPW = B // _NW
_CHUNK = 128
_NCH = _BPW // _CHUNK

_BLK = 4096
_NBLK = B // _BLK

_TCOL = 16384
_NLO = NPAIR // _TCOL
_TGRID = _NLO + 1
_TAILSTART = (N_NODE // _TCOL) * _TCOL
_TAILN = N_NODE - _TAILSTART
_NHIB = _TAILSTART // _TCOL - 1
PTAB = _TGRID * _TCOL


def _densify_body(lo_ref, hi_ref, tail_ref, out_ref):
    k = pl.program_id(0)

    @pl.when(k < _NLO)
    def _main():
        out_ref[...] = jnp.concatenate([lo_ref[...].T, hi_ref[...].T],
                                       axis=1)

    @pl.when(k == _NLO)
    def _tail():
        out_ref[pl.ds(0, _TAILN), pl.ds(0, EMD)] = tail_ref[...]


def _densify(tableT, tail):
    return pl.pallas_call(
        _densify_body,
        grid=(_TGRID,),
        in_specs=[
            pl.BlockSpec((EMD, _TCOL),
                         lambda k: (0, jnp.minimum(k, _NLO - 1))),
            pl.BlockSpec((EMD, _TCOL),
                         lambda k: (0, jnp.minimum(k + _NLO, _NHIB))),
            pl.BlockSpec((_TAILN, EMD), lambda k: (0, 0)),
        ],
        out_specs=pl.BlockSpec((_TCOL, 2 * EMD), lambda k: (k, 0)),
        out_shape=jax.ShapeDtypeStruct((PTAB, 2 * EMD), jnp.float32),
        compiler_params=pltpu.CompilerParams(
            dimension_semantics=("arbitrary",),
        ),
    )(tableT, tableT, tail)


def _gather_body(ids_hbm, table_hbm, out_hbm, idx_v, rows_v, sem):
    wid = lax.axis_index("s") * _NC + lax.axis_index("c")
    base = wid * _BPW
    pltpu.sync_copy(ids_hbm.at[wid], idx_v)
    cps = [
        pltpu.async_copy(
            table_hbm.at[idx_v.at[j]],
            rows_v.at[pl.ds(j * _CHUNK, _CHUNK)],
            sem,
        )
        for j in range(_NCH)
    ]
    for cp in cps:
        cp.wait()
    pltpu.sync_copy(rows_v, out_hbm.at[pl.ds(base, _BPW)])


@functools.cache
def _sc_gather_fn():
    return functools.partial(
        pl.kernel,
        out_type=jax.ShapeDtypeStruct((B, 2 * EMD), jnp.float32),
        mesh=plsc.VectorSubcoreMesh(
            core_axis_name="c", subcore_axis_name="s", num_cores=_NC),
        scratch_types=[
            pltpu.VMEM((_NCH, _CHUNK), jnp.int32),
            pltpu.VMEM((_BPW, 2 * EMD), jnp.float32),
            pltpu.SemaphoreType.DMA,
        ],
    )(_gather_body)


def _dense_body(pairs_ref, par_ref, noise_ref, dis_ref, w_ref, b_ref,
                fake0_ref, fake1_ref, sc_ref, acc_ref):
    k = pl.program_id(0)

    @pl.when(k == 0)
    def _init():
        acc_ref[0] = 0.0
        acc_ref[1] = 0.0
        acc_ref[2] = 0.0

    pairs = pairs_ref[...]
    par = par_ref[...]
    lane = lax.broadcasted_iota(jnp.int32, (_BLK, 2 * EMD), 1)
    keep = (lane < EMD) == (par < 0.5)
    m = jnp.where(keep, pairs, 0.0)
    partial_emb = jnp.sum(m * m)

    ce = []
    for i in range(2):
        w2 = jnp.concatenate([w_ref[i], w_ref[i]], axis=0)
        embw = jnp.dot(m, w2, preferred_element_type=jnp.float32)
        noisew = lax.dot_general(noise_ref[i], w_ref[i],
                                 (((0,), (0,)), ((), ())),
                                 preferred_element_type=jnp.float32)
        fake = embw + noisew + b_ref[i]
        fake = jnp.where(fake >= 0, fake, 0.2 * fake)
        faket = fake.T
        if i == 0:
            fake0_ref[...] = faket
        else:
            fake1_ref[...] = faket
        score = jnp.sum(dis_ref[i] * faket, axis=0, keepdims=True)
        ce_el = (jnp.maximum(score, 0.0) - score * (1.0 - LABEL_SMOOTH)
                 + jnp.log(1.0 + jnp.exp(-jnp.abs(score))))
        ce.append(jnp.sum(ce_el))

    acc_ref[0] = acc_ref[0] + ce[0]
    acc_ref[1] = acc_ref[1] + ce[1]
    acc_ref[2] = acc_ref[2] + partial_emb

    @pl.when(k == _NBLK - 1)
    def _fin():
        semb = acc_ref[2]
        w0 = w_ref[0]
        w1 = w_ref[1]
        n0 = (acc_ref[0] / B
              + LAMBDA_GEN * (0.5 * semb + 0.5 * jnp.sum(w0 * w0)))
        n1 = (acc_ref[1] / B
              + LAMBDA_GEN * (0.5 * semb + 0.5 * jnp.sum(w1 * w1)))
        sc_ref[0] = n0 + n1
        sc_ref[1] = n0
        sc_ref[2] = n1


def _dense(pairs, par, noise, dis, w, b3, interpret=False):
    return pl.pallas_call(
        _dense_body,
        grid=(_NBLK,),
        in_specs=[
            pl.BlockSpec((_BLK, 2 * EMD), lambda k: (k, 0)),
            pl.BlockSpec((_BLK, 1), lambda k: (k, 0)),
            pl.BlockSpec((2, EMD, _BLK), lambda k: (0, 0, k)),
            pl.BlockSpec((2, EMD, _BLK), lambda k: (0, 0, k)),
            pl.BlockSpec((2, EMD, EMD), lambda k: (0, 0, 0)),
            pl.BlockSpec((2, 1, EMD), lambda k: (0, 0, 0)),
        ],
        out_specs=[
            pl.BlockSpec((EMD, _BLK), lambda k: (0, k)),
            pl.BlockSpec((EMD, _BLK), lambda k: (0, k)),
            pl.BlockSpec(memory_space=pltpu.SMEM),
        ],
        out_shape=[
            jax.ShapeDtypeStruct((EMD, B), jnp.float32),
            jax.ShapeDtypeStruct((EMD, B), jnp.float32),
            jax.ShapeDtypeStruct((3,), jnp.float32),
        ],
        scratch_shapes=[pltpu.SMEM((3,), jnp.float32)],
        compiler_params=pltpu.CompilerParams(
            dimension_semantics=("arbitrary",),
        ),
        interpret=interpret,
    )(pairs, par, noise, dis, w, b3)


def kernel(node_ids, noise_embedding, dis_node_embedding, table, gen_w_1,
           gen_b_1):
    ids = node_ids.astype(jnp.int32)
    tail = ids >= _TAILSTART
    hi = (ids >= NPAIR) & ~tail
    pid = jnp.where(tail, ids - _TAILSTART + NPAIR,
                    jnp.where(hi, ids - NPAIR, ids))
    pair_ids = pid.reshape(_NW, _NCH, _CHUNK)
    par = hi.astype(jnp.float32).reshape(B, 1)
    pair_table = _densify(
        table.T, lax.slice(table, (_TAILSTART, 0), (N_NODE, EMD)))
    pairs = _sc_gather_fn()(pair_ids, pair_table)
    b3 = gen_b_1.reshape(2, 1, EMD)
    fake0t, fake1t, sc = _dense(pairs, par,
                                jnp.swapaxes(noise_embedding, 1, 2),
                                jnp.swapaxes(dis_node_embedding, 1, 2),
                                gen_w_1, b3)
    return (sc[0], fake0t.T, fake1t.T, sc[1], sc[2])


# speedup vs baseline: 2.6005x; 1.0008x over previous
import functools

import jax
import jax.numpy as jnp
from jax import lax
from jax.experimental import pallas as pl
from jax.experimental.pallas import tpu as pltpu
from jax.experimental.pallas import tpu_sc as plsc

N_NODE = 1000000
EMD = 64
B = 16384
LABEL_SMOOTH = 0.1
LAMBDA_GEN = 1e-05

NPAIR = 524288

_NC = 2
_NS = 16
_NW = _NC * _NS
_BPW = B // _NW
_CHUNK = 128
_NCH = _BPW // _CHUNK

_BLK = 4096
_NBLK = B // _BLK

_TCOL = 16384
_NLO = NPAIR // _TCOL
_TGRID = _NLO + 1
_TAILSTART = (N_NODE // _TCOL) * _TCOL
_TAILN = N_NODE - _TAILSTART
_NHIB = _TAILSTART // _TCOL - 1
PTAB = _TGRID * _TCOL


def _densify_body(lo_ref, hi_ref, tail_ref, out_ref):
    k = pl.program_id(0)

    @pl.when(k < _NLO)
    def _main():
        out_ref[...] = jnp.concatenate([lo_ref[...].T, hi_ref[...].T],
                                       axis=1)

    @pl.when(k == _NLO)
    def _tail():
        out_ref[pl.ds(0, _TAILN), pl.ds(0, EMD)] = tail_ref[...]


def _densify(tableT, tail):
    return pl.pallas_call(
        _densify_body,
        grid=(_TGRID,),
        in_specs=[
            pl.BlockSpec((EMD, _TCOL),
                         lambda k: (0, jnp.minimum(k, _NLO - 1))),
            pl.BlockSpec((EMD, _TCOL),
                         lambda k: (0, jnp.minimum(k + _NLO, _NHIB))),
            pl.BlockSpec((_TAILN, EMD), lambda k: (0, 0)),
        ],
        out_specs=pl.BlockSpec((_TCOL, 2 * EMD), lambda k: (k, 0)),
        out_shape=jax.ShapeDtypeStruct((PTAB, 2 * EMD), jnp.float32),
        compiler_params=pltpu.CompilerParams(
            dimension_semantics=("arbitrary",),
        ),
    )(tableT, tableT, tail)


def _gather_body(ids_hbm, table_hbm, out_hbm, idx_v, rows_v, sem, sem2):
    wid = lax.axis_index("s") * _NC + lax.axis_index("c")
    base = wid * _BPW
    pltpu.sync_copy(ids_hbm.at[wid], idx_v)
    cps = [
        pltpu.async_copy(
            table_hbm.at[idx_v.at[j]],
            rows_v.at[pl.ds(j * _CHUNK, _CHUNK)],
            sem,
        )
        for j in range(_NCH)
    ]
    scs = []
    for j in range(_NCH):
        cps[j].wait()
        scs.append(pltpu.async_copy(
            rows_v.at[pl.ds(j * _CHUNK, _CHUNK)],
            out_hbm.at[pl.ds(base + j * _CHUNK, _CHUNK)],
            sem2,
        ))
    for s in scs:
        s.wait()


@functools.cache
def _sc_gather_fn():
    return functools.partial(
        pl.kernel,
        out_type=jax.ShapeDtypeStruct((B, 2 * EMD), jnp.float32),
        mesh=plsc.VectorSubcoreMesh(
            core_axis_name="c", subcore_axis_name="s", num_cores=_NC),
        scratch_types=[
            pltpu.VMEM((_NCH, _CHUNK), jnp.int32),
            pltpu.VMEM((_BPW, 2 * EMD), jnp.float32),
            pltpu.SemaphoreType.DMA,
            pltpu.SemaphoreType.DMA,
        ],
    )(_gather_body)


def _dense_body(pairs_ref, par_ref, noise_ref, dis_ref, w_ref, b_ref,
                fake0_ref, fake1_ref, sc_ref, acc_ref):
    k = pl.program_id(0)

    @pl.when(k == 0)
    def _init():
        acc_ref[0] = 0.0
        acc_ref[1] = 0.0
        acc_ref[2] = 0.0

    pairs = pairs_ref[...]
    par = par_ref[...]
    lane = lax.broadcasted_iota(jnp.int32, (_BLK, 2 * EMD), 1)
    keep = (lane < EMD) == (par < 0.5)
    m = jnp.where(keep, pairs, 0.0)
    partial_emb = jnp.sum(m * m)

    ce = []
    for i in range(2):
        w2 = jnp.concatenate([w_ref[i], w_ref[i]], axis=0)
        embw = jnp.dot(m, w2, preferred_element_type=jnp.float32)
        noisew = lax.dot_general(noise_ref[i], w_ref[i],
                                 (((0,), (0,)), ((), ())),
                                 preferred_element_type=jnp.float32)
        fake = embw + noisew + b_ref[i]
        fake = jnp.where(fake >= 0, fake, 0.2 * fake)
        faket = fake.T
        if i == 0:
            fake0_ref[...] = faket
        else:
            fake1_ref[...] = faket
        score = jnp.sum(dis_ref[i] * faket, axis=0, keepdims=True)
        ce_el = (jnp.maximum(score, 0.0) - score * (1.0 - LABEL_SMOOTH)
                 + jnp.log(1.0 + jnp.exp(-jnp.abs(score))))
        ce.append(jnp.sum(ce_el))

    acc_ref[0] = acc_ref[0] + ce[0]
    acc_ref[1] = acc_ref[1] + ce[1]
    acc_ref[2] = acc_ref[2] + partial_emb

    @pl.when(k == _NBLK - 1)
    def _fin():
        semb = acc_ref[2]
        w0 = w_ref[0]
        w1 = w_ref[1]
        n0 = (acc_ref[0] / B
              + LAMBDA_GEN * (0.5 * semb + 0.5 * jnp.sum(w0 * w0)))
        n1 = (acc_ref[1] / B
              + LAMBDA_GEN * (0.5 * semb + 0.5 * jnp.sum(w1 * w1)))
        sc_ref[0] = n0 + n1
        sc_ref[1] = n0
        sc_ref[2] = n1


def _dense(pairs, par, noise, dis, w, b3, interpret=False):
    return pl.pallas_call(
        _dense_body,
        grid=(_NBLK,),
        in_specs=[
            pl.BlockSpec((_BLK, 2 * EMD), lambda k: (k, 0)),
            pl.BlockSpec((_BLK, 1), lambda k: (k, 0)),
            pl.BlockSpec((2, EMD, _BLK), lambda k: (0, 0, k)),
            pl.BlockSpec((2, EMD, _BLK), lambda k: (0, 0, k)),
            pl.BlockSpec((2, EMD, EMD), lambda k: (0, 0, 0)),
            pl.BlockSpec((2, 1, EMD), lambda k: (0, 0, 0)),
        ],
        out_specs=[
            pl.BlockSpec((EMD, _BLK), lambda k: (0, k)),
            pl.BlockSpec((EMD, _BLK), lambda k: (0, k)),
            pl.BlockSpec(memory_space=pltpu.SMEM),
        ],
        out_shape=[
            jax.ShapeDtypeStruct((EMD, B), jnp.float32),
            jax.ShapeDtypeStruct((EMD, B), jnp.float32),
            jax.ShapeDtypeStruct((3,), jnp.float32),
        ],
        scratch_shapes=[pltpu.SMEM((3,), jnp.float32)],
        compiler_params=pltpu.CompilerParams(
            dimension_semantics=("arbitrary",),
        ),
        interpret=interpret,
    )(pairs, par, noise, dis, w, b3)


def kernel(node_ids, noise_embedding, dis_node_embedding, table, gen_w_1,
           gen_b_1):
    ids = node_ids.astype(jnp.int32)
    tail = ids >= _TAILSTART
    hi = (ids >= NPAIR) & ~tail
    pid = jnp.where(tail, ids - _TAILSTART + NPAIR,
                    jnp.where(hi, ids - NPAIR, ids))
    pair_ids = pid.reshape(_NW, _NCH, _CHUNK)
    par = hi.astype(jnp.float32).reshape(B, 1)
    pair_table = _densify(
        table.T, lax.slice(table, (_TAILSTART, 0), (N_NODE, EMD)))
    pairs = _sc_gather_fn()(pair_ids, pair_table)
    b3 = gen_b_1.reshape(2, 1, EMD)
    fake0t, fake1t, sc = _dense(pairs, par,
                                jnp.swapaxes(noise_embedding, 1, 2),
                                jnp.swapaxes(dis_node_embedding, 1, 2),
                                gen_w_1, b3)
    return (sc[0], fake0t.T, fake1t.T, sc[1], sc[2])
